# baseline (speedup 1.0000x reference)
import jax
import jax.numpy as jnp
from jax import lax
from jax.experimental import pallas as pl
from jax.experimental.pallas import tpu as pltpu
from jax.experimental.pallas import tpu_sc as plsc

L = 16
NC, NS = 2, 16
NW = NC * NS
CHUNK = 80


def _dot_body(h_hbm, src_hbm, dst_hbm, out_hbm,
              h_sh, idx_u_all, idx_v_all, u0, u1, v0, v1, out_all, part,
              su0, su1, sv0, sv1):
    e_total = out_hbm.shape[0]
    d = h_hbm.shape[1]
    c = CHUNK
    epw = e_total // NW
    n_chunks = epw // c

    wid = lax.axis_index("s") * NC + lax.axis_index("c")
    wbase = wid * epw

    ubufs, vbufs = [u0, u1], [v0, v1]
    usems, vsems = [su0, su1], [sv0, sv1]

    cpu = pltpu.async_copy(src_hbm.at[pl.ds(wbase, epw)], idx_u_all, su0)
    cpv = pltpu.async_copy(dst_hbm.at[pl.ds(wbase, epw)], idx_v_all, sv0)

    @pl.when(lax.axis_index("s") == 0)
    def _():
        pltpu.sync_copy(h_hbm, h_sh)

    cpu.wait()
    cpv.wait()
    plsc.subcore_barrier()

    def issue(ci, b):
        pltpu.async_copy(
            h_sh.at[idx_u_all.at[pl.ds(ci * c, c)]], ubufs[b], usems[b])
        pltpu.async_copy(
            h_sh.at[idx_v_all.at[pl.ds(ci * c, c)]], vbufs[b], vsems[b])

    def wait(b):
        pltpu.make_async_copy(
            h_sh.at[idx_u_all.at[pl.ds(0, c)]], ubufs[b], usems[b]).wait()
        pltpu.make_async_copy(
            h_sh.at[idx_v_all.at[pl.ds(0, c)]], vbufs[b], vsems[b]).wait()

    nk = d // L
    unroll = 6

    def compute(ci, b):
        u_rows, v_rows = ubufs[b], vbufs[b]

        @plsc.parallel_loop(0, c, unroll=unroll)
        def _edge_body(e):
            prods = []
            for k in range(nk):
                uu = plsc.bitcast(u_rows[e, pl.ds(k * L, L)], jnp.bfloat16)
                vv = plsc.bitcast(v_rows[e, pl.ds(k * L, L)], jnp.bfloat16)
                prods.append(uu * vv)
            while len(prods) > 1:
                prods = [a + b2 for a, b2 in zip(prods[::2], prods[1::2])]
            p0, p1 = plsc.unpack(prods[0], format=plsc.PackFormat.INTERLEAVED)
            part[e, pl.ds(0, L)] = p0 + p1

        @plsc.parallel_loop(0, c // L, unroll=2)
        def _group_body(g):
            rows = g * L + lax.iota(jnp.int32, L)
            acc = jnp.zeros((L,), jnp.float32)
            for i in range(L):
                col = jnp.full((L,), i, jnp.int32)
                acc = acc + plsc.load_gather(part, [rows, col])
            out_all[pl.ds(ci * c + g * L, L)] = acc

    issue(0, 0)

    def pair_body(p, carry):
        for b in range(2):
            ci = p * 2 + b

            @pl.when(ci + 1 < n_chunks)
            def _():
                issue(ci + 1, 1 - b)

            wait(b)
            compute(ci, b)
        return carry

    lax.fori_loop(0, n_chunks // 2, pair_body, 0)
    if n_chunks % 2:
        wait(0)
        compute(n_chunks - 1, 0)

    pltpu.sync_copy(out_all, out_hbm.at[pl.ds(wbase, epw)])


def kernel(h, edge_index):
    e_total = edge_index.shape[1]
    d = h.shape[1]
    epw = e_total // NW
    src = edge_index[0].astype(jnp.int32)
    dst = edge_index[1].astype(jnp.int32)
    h = jax.lax.bitcast_convert_type(
        h.astype(jnp.bfloat16).reshape(h.shape[0], d // 2, 2), jnp.int32)

    sc_call = pl.kernel(
        _dot_body,
        out_type=jax.ShapeDtypeStruct((e_total,), jnp.float32),
        mesh=plsc.VectorSubcoreMesh(core_axis_name="c", subcore_axis_name="s"),
        scratch_types=[
            pltpu.VMEM_SHARED((h.shape[0], d // 2), jnp.int32),
            pltpu.VMEM((epw,), jnp.int32),
            pltpu.VMEM((epw,), jnp.int32),
            pltpu.VMEM((CHUNK, d // 2), jnp.int32),
            pltpu.VMEM((CHUNK, d // 2), jnp.int32),
            pltpu.VMEM((CHUNK, d // 2), jnp.int32),
            pltpu.VMEM((CHUNK, d // 2), jnp.int32),
            pltpu.VMEM((epw,), jnp.float32),
            pltpu.VMEM((CHUNK, 17), jnp.float32),
            pltpu.SemaphoreType.DMA,
            pltpu.SemaphoreType.DMA,
            pltpu.SemaphoreType.DMA,
            pltpu.SemaphoreType.DMA,
        ],
        compiler_params=pltpu.CompilerParams(
            needs_layout_passes=False, use_tc_tiling_on_sc=False),
    )
    out = sc_call(h, src, dst)
    return out.reshape(e_total, 1)


# speedup vs baseline: 1.0185x; 1.0185x over previous
import jax
import jax.numpy as jnp
from jax import lax
from jax.experimental import pallas as pl
from jax.experimental.pallas import tpu as pltpu
from jax.experimental.pallas import tpu_sc as plsc

L = 16
NC, NS = 2, 16
NW = NC * NS
CHUNK = 80


def _dot_body(h_hbm, src_hbm, dst_hbm, out_hbm,
              h_sh, idx_u_all, idx_v_all, u0, u1, v0, v1, out_all, part,
              su0, su1, sv0, sv1):
    e_total = out_hbm.shape[0]
    d = h_hbm.shape[1]
    c = CHUNK
    epw = e_total // NW
    n_chunks = epw // c

    wid = lax.axis_index("s") * NC + lax.axis_index("c")
    wbase = wid * epw

    ubufs, vbufs = [u0, u1], [v0, v1]
    usems, vsems = [su0, su1], [sv0, sv1]

    cpu = pltpu.async_copy(src_hbm.at[pl.ds(wbase, epw)], idx_u_all, su0)
    cpv = pltpu.async_copy(dst_hbm.at[pl.ds(wbase, epw)], idx_v_all, sv0)

    @pl.when(lax.axis_index("s") == 0)
    def _():
        pltpu.sync_copy(h_hbm, h_sh)

    cpu.wait()
    cpv.wait()
    plsc.subcore_barrier()

    def issue(ci, b):
        pltpu.async_copy(
            h_sh.at[idx_u_all.at[pl.ds(ci * c, c)]], ubufs[b], usems[b])
        pltpu.async_copy(
            h_sh.at[idx_v_all.at[pl.ds(ci * c, c)]], vbufs[b], vsems[b])

    def wait(b):
        pltpu.make_async_copy(
            h_sh.at[idx_u_all.at[pl.ds(0, c)]], ubufs[b], usems[b]).wait()
        pltpu.make_async_copy(
            h_sh.at[idx_v_all.at[pl.ds(0, c)]], vbufs[b], vsems[b]).wait()

    nk = d // L
    unroll = 4

    def compute(ci, b):
        u_rows, v_rows = ubufs[b], vbufs[b]

        @plsc.parallel_loop(0, c, unroll=unroll)
        def _edge_body(e):
            prods = []
            for k in range(nk):
                uu = plsc.bitcast(u_rows[e, pl.ds(k * L, L)], jnp.bfloat16)
                vv = plsc.bitcast(v_rows[e, pl.ds(k * L, L)], jnp.bfloat16)
                prods.append(uu * vv)
            while len(prods) > 1:
                prods = [a + b2 for a, b2 in zip(prods[::2], prods[1::2])]
            p0, p1 = plsc.unpack(prods[0], format=plsc.PackFormat.INTERLEAVED)
            part[e, pl.ds(0, L)] = p0 + p1

        @plsc.parallel_loop(0, c // L, unroll=2)
        def _group_body(g):
            rows = g * L + lax.iota(jnp.int32, L)
            acc = jnp.zeros((L,), jnp.float32)
            for i in range(L):
                col = jnp.full((L,), i, jnp.int32)
                acc = acc + plsc.load_gather(part, [rows, col])
            out_all[pl.ds(ci * c + g * L, L)] = acc

    issue(0, 0)

    def pair_body(p, carry):
        for b in range(2):
            ci = p * 2 + b

            @pl.when(ci + 1 < n_chunks)
            def _():
                issue(ci + 1, 1 - b)

            wait(b)
            compute(ci, b)
        return carry

    lax.fori_loop(0, n_chunks // 2, pair_body, 0)
    if n_chunks % 2:
        wait(0)
        compute(n_chunks - 1, 0)

    pltpu.sync_copy(out_all, out_hbm.at[pl.ds(wbase, epw)])


def kernel(h, edge_index):
    e_total = edge_index.shape[1]
    d = h.shape[1]
    epw = e_total // NW
    src = edge_index[0].astype(jnp.int32)
    dst = edge_index[1].astype(jnp.int32)
    h = jax.lax.bitcast_convert_type(
        h.astype(jnp.bfloat16).reshape(h.shape[0], d // 2, 2), jnp.int32)

    sc_call = pl.kernel(
        _dot_body,
        out_type=jax.ShapeDtypeStruct((e_total,), jnp.float32),
        mesh=plsc.VectorSubcoreMesh(core_axis_name="c", subcore_axis_name="s"),
        scratch_types=[
            pltpu.VMEM_SHARED((h.shape[0], d // 2), jnp.int32),
            pltpu.VMEM((epw,), jnp.int32),
            pltpu.VMEM((epw,), jnp.int32),
            pltpu.VMEM((CHUNK, d // 2), jnp.int32),
            pltpu.VMEM((CHUNK, d // 2), jnp.int32),
            pltpu.VMEM((CHUNK, d // 2), jnp.int32),
            pltpu.VMEM((CHUNK, d // 2), jnp.int32),
            pltpu.VMEM((epw,), jnp.float32),
            pltpu.VMEM((CHUNK, 17), jnp.float32),
            pltpu.SemaphoreType.DMA,
            pltpu.SemaphoreType.DMA,
            pltpu.SemaphoreType.DMA,
            pltpu.SemaphoreType.DMA,
        ],
        compiler_params=pltpu.CompilerParams(
            needs_layout_passes=False, use_tc_tiling_on_sc=False),
    )
    out = sc_call(h, src, dst)
    return out.reshape(e_total, 1)
